# Initial kernel scaffold; baseline (speedup 1.0000x reference)
import jax
import jax.numpy as jnp
from jax.experimental import pallas as pl


def kernel(x, edge_index, W1, b1, g1, be1, W2, b2, g2, be2, gd, bd):
    raise NotImplementedError("write your pallas kernel here")


# speedup vs baseline: 2.3404x; 2.3404x over previous
import functools

import jax
import jax.numpy as jnp
from jax import lax
from jax.experimental import pallas as pl
from jax.experimental.pallas import tpu as pltpu
from jax.experimental.pallas import tpu_sc as plsc

N = 10000
E = 160000
IN_F = 128
HID = 64
ENC = 16
EPS = 1e-5

NC = 2
NS = 16
NW = NC * NS
CHUNK = 128
EP = ((E + NW * CHUNK - 1) // (NW * CHUNK)) * (NW * CHUNK)
NCH = EP // (NW * CHUNK)
NROWS = 10240
TRASH = N
RPT = NROWS // NS

_PREC = jax.lax.Precision.DEFAULT


def _make_seg_sum(feat):
    nvec = feat // 16

    mesh = plsc.VectorSubcoreMesh(core_axis_name="c", subcore_axis_name="s")

    @functools.partial(
        pl.kernel,
        out_type=jax.ShapeDtypeStruct((NC, NROWS, feat), jnp.float32),
        mesh=mesh,
        scratch_types=[
            pltpu.VMEM((NCH, CHUNK), jnp.int32),
            pltpu.VMEM((NCH, CHUNK), jnp.int32),
            pltpu.VMEM((CHUNK, feat), jnp.float32),
            pltpu.VMEM_SHARED((NROWS, feat), jnp.float32),
            pltpu.SemaphoreType.DMA,
        ],
    )
    def seg(x_hbm, src_hbm, dst_hbm, out_hbm, src_v, dst_v, rows_v, acc_sh, sem):
        c = lax.axis_index("c")
        s = lax.axis_index("s")
        wid = c * NS + s

        pltpu.sync_copy(src_hbm.at[wid], src_v)
        pltpu.sync_copy(dst_hbm.at[wid], dst_v)

        def zb(k, _):
            i = k // nvec
            j = k - i * nvec
            rows_v[i, pl.ds(j * 16, 16)] = jnp.zeros((16,), jnp.float32)
            return 0

        lax.fori_loop(0, CHUNK * nvec, zb, 0)
        for k in range(RPT // CHUNK):
            pltpu.sync_copy(rows_v, acc_sh.at[pl.ds(s * RPT + k * CHUNK, CHUNK)])
        plsc.subcore_barrier()

        def body(j, _):
            pltpu.async_copy(x_hbm.at[src_v.at[j]], rows_v, sem).wait()
            pltpu.sync_copy(rows_v, acc_sh.at[dst_v.at[j]], add=True)
            return 0

        lax.fori_loop(0, NCH, body, 0)
        plsc.subcore_barrier()

        pltpu.sync_copy(acc_sh.at[pl.ds(s * RPT, RPT)],
                        out_hbm.at[c, pl.ds(s * RPT, RPT)])

    return seg


_seg_sum_128 = _make_seg_sum(IN_F)


def _make_dense_body(in_dim, out_dim, pad_to):
    def body(p_ref, w_ref, b_ref, g_ref, be_ref, o_ref):
        sm = p_ref[0, :N, :in_dim] + p_ref[1, :N, :in_dim]
        t = lax.dot_general(sm, w_ref[...], (((1,), (0,)), ((), ())),
                            preferred_element_type=jnp.float32, precision=_PREC)
        t = jnp.tanh(t + b_ref[...])
        mean = jnp.mean(t, axis=0, keepdims=True)
        var = jnp.mean((t - mean) ** 2, axis=0, keepdims=True)
        r = (t - mean) / jnp.sqrt(var + EPS) * g_ref[...] + be_ref[...]
        if pad_to > out_dim:
            o_ref[:, :out_dim] = r
            o_ref[:, out_dim:] = jnp.zeros((N, pad_to - out_dim), jnp.float32)
        else:
            o_ref[...] = r
    return body


def _dense(p, w, b, g, be, in_dim, out_dim, pad_to):
    return pl.pallas_call(
        _make_dense_body(in_dim, out_dim, pad_to),
        out_shape=jax.ShapeDtypeStruct((N, pad_to), jnp.float32),
    )(p, w, b.reshape(1, -1), g.reshape(1, -1), be.reshape(1, -1))


BI = 400
NI = N // BI


def _dstats_body(zi_ref, z_ref, st_ref):
    d = lax.dot_general(zi_ref[...], z_ref[...], (((1,), (1,)), ((), ())),
                        preferred_element_type=jnp.float32, precision=_PREC)
    d = jax.nn.sigmoid(d)

    @pl.when(pl.program_id(0) == 0)
    def _():
        st_ref[...] = jnp.zeros_like(st_ref)

    st_ref[0:1, :] += jnp.sum(d, axis=0, keepdims=True)
    st_ref[1:2, :] += jnp.sum(d * d, axis=0, keepdims=True)


def _dnorm_body(zi_ref, z_ref, st_ref, gd_ref, bd_ref, o_ref):
    d = lax.dot_general(zi_ref[...], z_ref[...], (((1,), (1,)), ((), ())),
                        preferred_element_type=jnp.float32, precision=_PREC)
    d = jax.nn.sigmoid(d)
    mean = st_ref[0:1, :] * (1.0 / N)
    var = st_ref[1:2, :] * (1.0 / N) - mean * mean
    o_ref[...] = (d - mean) / jnp.sqrt(var + EPS) * gd_ref[...] + bd_ref[...]


def _decode(z, gd, bd):
    stats = pl.pallas_call(
        _dstats_body,
        grid=(NI,),
        in_specs=[
            pl.BlockSpec((BI, ENC), lambda i: (i, 0)),
            pl.BlockSpec((N, ENC), lambda i: (0, 0)),
        ],
        out_specs=pl.BlockSpec((2, N), lambda i: (0, 0)),
        out_shape=jax.ShapeDtypeStruct((2, N), jnp.float32),
    )(z, z)
    return pl.pallas_call(
        _dnorm_body,
        grid=(NI,),
        in_specs=[
            pl.BlockSpec((BI, ENC), lambda i: (i, 0)),
            pl.BlockSpec((N, ENC), lambda i: (0, 0)),
            pl.BlockSpec((2, N), lambda i: (0, 0)),
            pl.BlockSpec((1, N), lambda i: (0, 0)),
            pl.BlockSpec((1, N), lambda i: (0, 0)),
        ],
        out_specs=pl.BlockSpec((BI, N), lambda i: (i, 0)),
        out_shape=jax.ShapeDtypeStruct((N, N), jnp.float32),
    )(z, z, stats, gd.reshape(1, N), bd.reshape(1, N))


def kernel(x, edge_index, W1, b1, g1, be1, W2, b2, g2, be2, gd, bd):
    src = edge_index[0]
    dst = edge_index[1]
    pad = EP - E
    srcp = jnp.concatenate([src, jnp.zeros((pad,), jnp.int32)]).reshape(NW, NCH, CHUNK)
    dstp = jnp.concatenate([dst, jnp.full((pad,), TRASH, jnp.int32)]).reshape(NW, NCH, CHUNK)

    p1 = _seg_sum_128(x, srcp, dstp)
    h = _dense(p1, W1, b1, g1, be1, IN_F, HID, IN_F)
    p2 = _seg_sum_128(h, srcp, dstp)
    z = _dense(p2, W2, b2, g2, be2, HID, ENC, ENC)
    d = _decode(z, gd, bd)
    return (d, z)
